# Initial kernel scaffold; baseline (speedup 1.0000x reference)
import jax
import jax.numpy as jnp
from jax.experimental import pallas as pl


def kernel(x, table):
    raise NotImplementedError("write your pallas kernel here")


# speedup vs baseline: 5.2069x; 5.2069x over previous
import functools

import jax
import jax.numpy as jnp
from jax import lax
from jax.experimental import pallas as pl
from jax.experimental.pallas import tpu as pltpu
from jax.experimental.pallas import tpu_sc as plsc

_NC, _NS, _L = 2, 16, 16
_NW = _NC * _NS


def _emb_body(C, R, n_chunks, x_hbm, tab_hbm, out_hbm, idx_v, out_v, tab_v):
    wid = lax.axis_index("s") * _NC + lax.axis_index("c")
    pltpu.sync_copy(tab_hbm, tab_v)
    iota_r = lax.iota(jnp.int32, _L) * R

    def chunk(g, carry):
        base = wid * (C * n_chunks) + g * C
        pltpu.sync_copy(x_hbm.at[pl.ds(base, C)], idx_v)

        def group(i, inner_carry):
            iv = idx_v[pl.ds(i * _L, _L)] * R
            for c in range(R):
                col = plsc.load_gather(tab_v, [iv + c])
                plsc.store_scatter(out_v, [iota_r + (i * (_L * R) + c)], col)
            return inner_carry

        lax.fori_loop(0, C // _L, group, 0)

        pltpu.sync_copy(out_v, out_hbm.at[pl.ds(base * R, C * R)])
        return carry

    lax.fori_loop(0, n_chunks, chunk, 0)


def _pick_chunk(per_w):
    for c in range(min(per_w, 16384), 0, -16):
        if per_w % c == 0:
            return c
    return per_w


def kernel(x, table):
    B, S = x.shape
    V, R = table.shape
    N = B * S
    assert N % _NW == 0
    per_w = N // _NW
    C = _pick_chunk(per_w)
    n_chunks = per_w // C

    body = functools.partial(_emb_body, C, R, n_chunks)
    k = pl.kernel(
        body,
        out_type=jax.ShapeDtypeStruct((N * R,), table.dtype),
        compiler_params=pltpu.CompilerParams(needs_layout_passes=False),
        mesh=plsc.VectorSubcoreMesh(
            core_axis_name="c", subcore_axis_name="s",
            num_cores=_NC, num_subcores=_NS,
        ),
        scratch_types=[
            pltpu.VMEM((C,), jnp.int32),
            pltpu.VMEM((C * R,), jnp.float32),
            pltpu.VMEM((V * R,), jnp.float32),
        ],
    )
    out = k(x.reshape(N).astype(jnp.int32), table.reshape(V * R))
    return out.reshape(B, S, R)


# speedup vs baseline: 5.2123x; 1.0010x over previous
import functools

import jax
import jax.numpy as jnp
from jax import lax
from jax.experimental import pallas as pl
from jax.experimental.pallas import tpu as pltpu
from jax.experimental.pallas import tpu_sc as plsc

_NC, _NS, _L = 2, 16, 16
_NW = _NC * _NS


def _emb_body(C, R, n_chunks, x_hbm, tab_hbm, out_hbm, idx_v, out_v, tab_v):
    wid = lax.axis_index("s") * _NC + lax.axis_index("c")
    pltpu.sync_copy(tab_hbm, tab_v)
    G = _L // R
    rep_pats = [jnp.arange(_L, dtype=jnp.int32) // R + q * G for q in range(R)]
    lane_mod = jnp.arange(_L, dtype=jnp.int32) % R

    def chunk(g, carry):
        base = wid * (C * n_chunks) + g * C
        pltpu.sync_copy(x_hbm.at[pl.ds(base, C)], idx_v)

        U = 8

        def group(i, inner_carry):
            for u in range(U):
                j = i * U + u
                iv = idx_v[pl.ds(j * _L, _L)] * R
                for q in range(R):
                    rep = jnp.take_along_axis(iv, rep_pats[q], axis=0)
                    val = plsc.load_gather(tab_v, [rep + lane_mod])
                    out_v[pl.ds((j * R + q) * _L, _L)] = val
            return inner_carry

        lax.fori_loop(0, C // (_L * U), group, 0)

        pltpu.sync_copy(out_v, out_hbm.at[pl.ds(base * R, C * R)])
        return carry

    lax.fori_loop(0, n_chunks, chunk, 0)


def _pick_chunk(per_w):
    for c in range(min(per_w, 16384), 0, -16):
        if per_w % c == 0:
            return c
    return per_w


def kernel(x, table):
    B, S = x.shape
    V, R = table.shape
    N = B * S
    assert N % _NW == 0
    per_w = N // _NW
    C = _pick_chunk(per_w)
    n_chunks = per_w // C

    body = functools.partial(_emb_body, C, R, n_chunks)
    k = pl.kernel(
        body,
        out_type=jax.ShapeDtypeStruct((N * R,), table.dtype),
        compiler_params=pltpu.CompilerParams(needs_layout_passes=False),
        mesh=plsc.VectorSubcoreMesh(
            core_axis_name="c", subcore_axis_name="s",
            num_cores=_NC, num_subcores=_NS,
        ),
        scratch_types=[
            pltpu.VMEM((C,), jnp.int32),
            pltpu.VMEM((C * R,), jnp.float32),
            pltpu.VMEM((V * R,), jnp.float32),
        ],
    )
    out = k(x.reshape(N).astype(jnp.int32), table.reshape(V * R))
    return out.reshape(B, S, R)


# speedup vs baseline: 60.6771x; 11.6411x over previous
import functools

import jax
import jax.numpy as jnp
from jax import lax
from jax.experimental import pallas as pl
from jax.experimental.pallas import tpu as pltpu
from jax.experimental.pallas import tpu_sc as plsc

_NC, _NS, _L = 2, 16, 16
_NW = _NC * _NS
_SL = 8


def _emb_body(R, NU, UPW, ICH, n_i, x_hbm, tab_hbm, out_hbm, in_v, out_v,
              tab_v):
    wid = lax.axis_index("s") * _NC + lax.axis_index("c")
    pltpu.sync_copy(tab_hbm, tab_v)

    def unit(t, carry):
        u = wid + t * _NW

        @pl.when(u < NU)
        def _():
            tj = u // n_i
            i0 = (u % n_i) * ICH
            pltpu.sync_copy(
                x_hbm.at[pl.ds(tj * _SL, _SL), pl.ds(i0, ICH)], in_v)
            for s in range(_SL):

                def grp(g, c2, s=s):
                    iv = in_v[s, pl.ds(g * _L, _L)] * R
                    for q in range(R):
                        val = plsc.load_gather(tab_v, [iv + q])
                        out_v[s, q, pl.ds(g * _L, _L)] = val
                    return c2

                lax.fori_loop(0, ICH // _L, grp, 0)
            pltpu.sync_copy(
                out_v, out_hbm.at[pl.ds(tj * _SL, _SL), :, pl.ds(i0, ICH)])

        return carry

    lax.fori_loop(0, UPW, unit, 0)


def kernel(x, table):
    B, S = x.shape
    V, R = table.shape
    ICH = 2048
    n_i = B // ICH
    NU = (S // _SL) * n_i
    UPW = -(-NU // _NW)

    body = functools.partial(_emb_body, R, NU, UPW, ICH, n_i)
    k = pl.kernel(
        body,
        out_type=jax.ShapeDtypeStruct((S, R, B), table.dtype),
        compiler_params=pltpu.CompilerParams(
            needs_layout_passes=False, use_tc_tiling_on_sc=True),
        mesh=plsc.VectorSubcoreMesh(
            core_axis_name="c", subcore_axis_name="s",
            num_cores=_NC, num_subcores=_NS,
        ),
        scratch_types=[
            pltpu.VMEM((_SL, ICH), jnp.int32),
            pltpu.VMEM((_SL, R, ICH), jnp.float32),
            pltpu.VMEM((V * R,), jnp.float32),
        ],
    )
    ot = k(jnp.transpose(x), table.reshape(V * R))
    return jnp.transpose(ot, (2, 0, 1))


# speedup vs baseline: 67.6415x; 1.1148x over previous
import functools

import jax
import jax.numpy as jnp
from jax import lax
from jax.experimental import pallas as pl
from jax.experimental.pallas import tpu as pltpu
from jax.experimental.pallas import tpu_sc as plsc

_NC, _NS, _L = 2, 16, 16
_NW = _NC * _NS
_SL = 8


def _emb_body(R, NU, UPW, ICH, n_i, x_hbm, tab_hbm, out_hbm, in_v, out_v,
              tab_v):
    wid = lax.axis_index("s") * _NC + lax.axis_index("c")
    pltpu.sync_copy(tab_hbm, tab_v)

    def unit(t, carry):
        u = wid + t * _NW

        @pl.when(u < NU)
        def _():
            tj = u // n_i
            i0 = (u % n_i) * ICH
            pltpu.sync_copy(
                x_hbm.at[pl.ds(tj * _SL, _SL), pl.ds(i0, ICH)], in_v)
            U = 8
            for s in range(_SL):

                def grp(g, c2, s=s):
                    for u in range(U):
                        j = g * U + u
                        iv = in_v[s, pl.ds(j * _L, _L)] * R
                        for q in range(R):
                            val = plsc.load_gather(tab_v, [iv + q])
                            out_v[s, q, pl.ds(j * _L, _L)] = val
                    return c2

                lax.fori_loop(0, ICH // (_L * U), grp, 0)
            pltpu.sync_copy(
                out_v, out_hbm.at[pl.ds(tj * _SL, _SL), :, pl.ds(i0, ICH)])

        return carry

    lax.fori_loop(0, UPW, unit, 0)


def kernel(x, table):
    B, S = x.shape
    V, R = table.shape
    ICH = 2048
    n_i = B // ICH
    NU = (S // _SL) * n_i
    UPW = -(-NU // _NW)

    body = functools.partial(_emb_body, R, NU, UPW, ICH, n_i)
    k = pl.kernel(
        body,
        out_type=jax.ShapeDtypeStruct((S, R, B), table.dtype),
        compiler_params=pltpu.CompilerParams(
            needs_layout_passes=False, use_tc_tiling_on_sc=True),
        mesh=plsc.VectorSubcoreMesh(
            core_axis_name="c", subcore_axis_name="s",
            num_cores=_NC, num_subcores=_NS,
        ),
        scratch_types=[
            pltpu.VMEM((_SL, ICH), jnp.int32),
            pltpu.VMEM((_SL, R, ICH), jnp.float32),
            pltpu.VMEM((V * R,), jnp.float32),
        ],
    )
    ot = k(jnp.transpose(x), table.reshape(V * R))
    return jnp.transpose(ot, (2, 0, 1))


# speedup vs baseline: 83.6083x; 1.2361x over previous
import functools

import jax
import jax.numpy as jnp
from jax import lax
from jax.experimental import pallas as pl
from jax.experimental.pallas import tpu as pltpu
from jax.experimental.pallas import tpu_sc as plsc

_NC, _NS, _L = 2, 16, 16
_NW = _NC * _NS
_SL = 8
_U = 8


def _emb_body(R, NT, ICH, x_hbm, tab_hbm, out_hbm,
              in0, in1, out0, out1, tab_v, sin0, sin1, sout0, sout1):
    wid = lax.axis_index("s") * _NC + lax.axis_index("c")
    i0 = wid * ICH
    pltpu.sync_copy(tab_hbm, tab_v)

    def in_slice(t):
        return x_hbm.at[pl.ds(t * _SL, _SL), pl.ds(i0, ICH)]

    def out_slice(t):
        return out_hbm.at[pl.ds(t * _SL, _SL), :, pl.ds(i0, ICH)]

    def compute(in_v, out_v):
        for s in range(_SL):

            def grp(g, c2, s=s):
                for u in range(_U):
                    j = g * _U + u
                    iv = in_v[s, pl.ds(j * _L, _L)] * R
                    for q in range(R):
                        val = plsc.load_gather(tab_v, [iv + q])
                        out_v[s, q, pl.ds(j * _L, _L)] = val
                return c2

            lax.fori_loop(0, ICH // (_L * _U), grp, 0)

    pltpu.async_copy(in_slice(0), in0, sin0)

    def pair(p, carry):
        t0 = p * 2

        pltpu.make_async_copy(in_slice(t0), in0, sin0).wait()

        @pl.when(t0 + 1 < NT)
        def _():
            pltpu.async_copy(in_slice(t0 + 1), in1, sin1)

        @pl.when(t0 >= 2)
        def _():
            pltpu.make_async_copy(out0, out_slice(t0 - 2), sout0).wait()

        compute(in0, out0)
        pltpu.async_copy(out0, out_slice(t0), sout0)

        @pl.when(t0 + 1 < NT)
        def _():
            t1 = t0 + 1
            pltpu.make_async_copy(in_slice(t1), in1, sin1).wait()

            @pl.when(t1 + 1 < NT)
            def _():
                pltpu.async_copy(in_slice(t1 + 1), in0, sin0)

            @pl.when(t1 >= 2)
            def _():
                pltpu.make_async_copy(out1, out_slice(t1 - 2), sout1).wait()

            compute(in1, out1)
            pltpu.async_copy(out1, out_slice(t1), sout1)

        return carry

    lax.fori_loop(0, (NT + 1) // 2, pair, 0)

    last_even = ((NT - 1) // 2) * 2
    pltpu.make_async_copy(out0, out_slice(last_even), sout0).wait()
    if NT > 1:
        last_odd = ((NT - 2) // 2) * 2 + 1
        pltpu.make_async_copy(out1, out_slice(last_odd), sout1).wait()


def kernel(x, table):
    B, S = x.shape
    V, R = table.shape
    ICH = B // _NW
    NT = S // _SL

    body = functools.partial(_emb_body, R, NT, ICH)
    k = pl.kernel(
        body,
        out_type=jax.ShapeDtypeStruct((S, R, B), table.dtype),
        compiler_params=pltpu.CompilerParams(
            needs_layout_passes=False, use_tc_tiling_on_sc=True),
        mesh=plsc.VectorSubcoreMesh(
            core_axis_name="c", subcore_axis_name="s",
            num_cores=_NC, num_subcores=_NS,
        ),
        scratch_types=[
            pltpu.VMEM((_SL, ICH), jnp.int32),
            pltpu.VMEM((_SL, ICH), jnp.int32),
            pltpu.VMEM((_SL, R, ICH), jnp.float32),
            pltpu.VMEM((_SL, R, ICH), jnp.float32),
            pltpu.VMEM((V * R,), jnp.float32),
            pltpu.SemaphoreType.DMA,
            pltpu.SemaphoreType.DMA,
            pltpu.SemaphoreType.DMA,
            pltpu.SemaphoreType.DMA,
        ],
    )
    ot = k(jnp.transpose(x), table.reshape(V * R))
    return jnp.transpose(ot, (2, 0, 1))


# speedup vs baseline: 218.9462x; 2.6187x over previous
import functools

import jax
import jax.numpy as jnp
from jax import lax
from jax.experimental import pallas as pl
from jax.experimental.pallas import tpu as pltpu
from jax.experimental.pallas import tpu_sc as plsc

_NC, _NS, _L = 2, 16, 16
_NW = _NC * _NS
_SL = 8
_U = 8


def _emb_body(R, NT, ICH, x_hbm, tab_hbm, out_hbm,
              in0, in1, out0, out1, tab_v, sin0, sin1, sout0, sout1):
    wid = lax.axis_index("s") * _NC + lax.axis_index("c")
    i0 = wid * ICH
    pltpu.sync_copy(tab_hbm, tab_v)

    def in_slice(t):
        return x_hbm.at[pl.ds(t * _SL, _SL), pl.ds(i0, ICH)]

    def out_slice(t):
        return out_hbm.at[pl.ds(t * _SL, _SL), :, pl.ds(i0, ICH)]

    def compute(in_v, out_v):
        for s in range(_SL):

            @plsc.parallel_loop(0, ICH // _L, unroll=_U)
            def _grp(g, s=s):
                iv = in_v[s, pl.ds(g * _L, _L)] * R
                for q in range(R):
                    val = plsc.load_gather(tab_v, [iv + q])
                    out_v[s, q, pl.ds(g * _L, _L)] = val

    pltpu.async_copy(in_slice(0), in0, sin0)

    def pair(p, carry):
        t0 = p * 2

        pltpu.make_async_copy(in_slice(t0), in0, sin0).wait()

        @pl.when(t0 + 1 < NT)
        def _():
            pltpu.async_copy(in_slice(t0 + 1), in1, sin1)

        @pl.when(t0 >= 2)
        def _():
            pltpu.make_async_copy(out0, out_slice(t0 - 2), sout0).wait()

        compute(in0, out0)
        pltpu.async_copy(out0, out_slice(t0), sout0)

        @pl.when(t0 + 1 < NT)
        def _():
            t1 = t0 + 1
            pltpu.make_async_copy(in_slice(t1), in1, sin1).wait()

            @pl.when(t1 + 1 < NT)
            def _():
                pltpu.async_copy(in_slice(t1 + 1), in0, sin0)

            @pl.when(t1 >= 2)
            def _():
                pltpu.make_async_copy(out1, out_slice(t1 - 2), sout1).wait()

            compute(in1, out1)
            pltpu.async_copy(out1, out_slice(t1), sout1)

        return carry

    lax.fori_loop(0, (NT + 1) // 2, pair, 0)

    last_even = ((NT - 1) // 2) * 2
    pltpu.make_async_copy(out0, out_slice(last_even), sout0).wait()
    if NT > 1:
        last_odd = ((NT - 2) // 2) * 2 + 1
        pltpu.make_async_copy(out1, out_slice(last_odd), sout1).wait()


def kernel(x, table):
    B, S = x.shape
    V, R = table.shape
    ICH = B // _NW
    NT = S // _SL

    body = functools.partial(_emb_body, R, NT, ICH)
    k = pl.kernel(
        body,
        out_type=jax.ShapeDtypeStruct((S, R, B), table.dtype),
        compiler_params=pltpu.CompilerParams(
            needs_layout_passes=False, use_tc_tiling_on_sc=True),
        mesh=plsc.VectorSubcoreMesh(
            core_axis_name="c", subcore_axis_name="s",
            num_cores=_NC, num_subcores=_NS,
        ),
        scratch_types=[
            pltpu.VMEM((_SL, ICH), jnp.int32),
            pltpu.VMEM((_SL, ICH), jnp.int32),
            pltpu.VMEM((_SL, R, ICH), jnp.float32),
            pltpu.VMEM((_SL, R, ICH), jnp.float32),
            pltpu.VMEM((V * R,), jnp.float32),
            pltpu.SemaphoreType.DMA,
            pltpu.SemaphoreType.DMA,
            pltpu.SemaphoreType.DMA,
            pltpu.SemaphoreType.DMA,
        ],
    )
    ot = k(jnp.transpose(x), table.reshape(V * R))
    return jnp.transpose(ot, (2, 0, 1))
